# baseline (speedup 1.0000x reference)
import functools

import jax
import jax.numpy as jnp
from jax import lax
from jax.experimental import pallas as pl
from jax.experimental.pallas import tpu as pltpu
from jax.experimental.pallas import tpu_sc as plsc

_NC = 2
_NS = 16
_NW = _NC * _NS

B, F, V, D = 16384, 26, 1000000, 32
TCOLS = (V + 127) // 128
BASE_CH = TCOLS // _NW
EXTRA = TCOLS - BASE_CH * _NW
A_ITERS = (BASE_CH + 1 + 1) // 2
BPW = B // _NW
CH = 128
NCHUNK = F * (BPW // CH)
VQ = TCOLS * 32


def _emb_call(flat_idx, tabT):
    mesh = plsc.VectorSubcoreMesh(core_axis_name="c", subcore_axis_name="s")

    @functools.partial(
        pl.kernel,
        out_type=jax.ShapeDtypeStruct((F, D, B), jnp.float32),
        mesh=mesh,
        compiler_params=pltpu.CompilerParams(
            use_tc_tiling_on_sc=True, needs_layout_passes=False),
        scratch_types=[
            pltpu.HBM((VQ, 128), jnp.float32),
            pltpu.VMEM((D, 128), jnp.float32),
            pltpu.VMEM((D, 128), jnp.float32),
            pltpu.VMEM((D, 128), jnp.float32),
            pltpu.VMEM((D, 128), jnp.float32),
            pltpu.VMEM((CH,), jnp.int32),
            pltpu.VMEM((CH,), jnp.int32),
            pltpu.VMEM((CH,), jnp.int32),
            pltpu.VMEM((CH,), jnp.int32),
            pltpu.VMEM((CH, 128), jnp.float32),
            pltpu.VMEM((CH, 128), jnp.float32),
            pltpu.VMEM((D, CH), jnp.float32),
            pltpu.VMEM((D, CH), jnp.float32),
            pltpu.SemaphoreType.DMA,
            pltpu.SemaphoreType.DMA,
            pltpu.SemaphoreType.DMA,
            pltpu.SemaphoreType.DMA,
            pltpu.SemaphoreType.DMA,
            pltpu.SemaphoreType.DMA,
            pltpu.SemaphoreType.DMA,
            pltpu.SemaphoreType.DMA,
            pltpu.SemaphoreType.DMA,
            pltpu.SemaphoreType.DMA,
            pltpu.HBM((16, 128), jnp.float32),
            pltpu.VMEM((8, 128), jnp.float32),
            pltpu.VMEM((8, 128), jnp.float32),
        ],
    )
    def emb(idx_hbm, tabT_hbm, outT_hbm, scratch,
            vin0, vin1, vtr0, vtr1, idx0, idx1, qid0, qid1,
            rows0, rows1, vout0, vout1,
            ain0, ain1, aout0, aout1, si0, si1, sg0, sg1, ss0, ss1,
            flags, fmark, fpoll):
        cid = lax.axis_index("c")
        sid = lax.axis_index("s")
        wid = sid * _NC + cid
        vin = [vin0, vin1]
        vtr = [vtr0, vtr1]
        idxb = [idx0, idx1]
        qidx = [qid0, qid1]
        rows = [rows0, rows1]
        vout = [vout0, vout1]
        ain = [ain0, ain1]
        aout = [aout0, aout1]
        sidx = [si0, si1]
        sgat = [sg0, sg1]
        ssto = [ss0, ss1]

        lane = lax.iota(jnp.int32, 16)

        my_flag = pl.multiple_of(cid * 8, 8)
        other_flag = pl.multiple_of((1 - cid) * 8, 8)

        @pl.when(sid == 0)
        def _():
            for r in range(8):
                for g in range(8):
                    fmark[r, pl.ds(g * 16, 16)] = jnp.zeros((16,), jnp.float32)
            pltpu.sync_copy(fmark, flags.at[pl.ds(my_flag, 8), :])

        count = BASE_CH + jnp.where(wid < EXTRA, 1, 0)
        base = wid * BASE_CH + jnp.minimum(wid, EXTRA)

        def col0(i):
            return pl.multiple_of((base + i) * 128, 128)

        def a_in(i, u):
            return pltpu.make_async_copy(
                tabT_hbm.at[:, pl.ds(col0(i), 128)], vin[u], ain[u])

        def a_out(i, u):
            return pltpu.make_async_copy(
                vtr[u],
                scratch.at[pl.ds(pl.multiple_of(col0(i) // 4, D), D), :],
                aout[u])

        a_in(0, 0).start()
        a_in(1, 1).start()

        def a_body(t, _):
            for u in (0, 1):
                i = 2 * t + u
                live = i < count

                @pl.when(live)
                def _():
                    a_in(i, u).wait()

                @pl.when(jnp.logical_and(i >= 2, live))
                def _():
                    a_out(i - 2, u).wait()

                @pl.when(live)
                def _():
                    for g in range(8):
                        rl = g * 16
                        qv = (lane + rl) >> 2
                        mv = ((lane + rl) & 3) * D
                        for e in range(D):
                            vals = vin[u][e, pl.ds(rl, 16)]
                            plsc.store_scatter(vtr[u], [qv, mv + e], vals)
                    a_out(i, u).start()

                @pl.when(i + 2 < count)
                def _():
                    a_in(i + 2, u).start()
            return ()

        lax.fori_loop(0, A_ITERS, a_body, (), unroll=False)
        a_out(0, 0).wait()
        a_out(0, 1).wait()

        plsc.subcore_barrier()

        @pl.when(sid == 0)
        def _():
            for r in range(8):
                for g in range(8):
                    fmark[r, pl.ds(g * 16, 16)] = jnp.full(
                        (16,), 12345678.0, jnp.float32)
            pltpu.sync_copy(fmark, flags.at[pl.ds(my_flag, 8), :])

            def poll_cond(seen):
                return seen == 0

            def poll_body(seen):
                pltpu.sync_copy(flags.at[pl.ds(other_flag, 8), :], fpoll)
                v = fpoll[0, pl.ds(0, 16)]
                return jnp.min(
                    (v == 12345678.0).astype(jnp.int32))

            lax.while_loop(poll_cond, poll_body, jnp.int32(0))

        plsc.subcore_barrier()

        b_base = wid * BPW

        def off(c):
            return c >> 2, pl.multiple_of(b_base + (c & 3) * CH, CH)

        def b_idx(c, u):
            f, bo = off(c)
            return pltpu.make_async_copy(
                idx_hbm.at[pl.ds(f * B + bo, CH)], idxb[u], sidx[u])

        def b_gather(c, u):
            return pltpu.make_async_copy(
                scratch.at[qidx[u]], rows[u], sgat[u])

        def b_store(c, u):
            f, bo = off(c)
            return pltpu.make_async_copy(
                vout[u], outT_hbm.at[f, :, pl.ds(bo, CH)], ssto[u])

        def make_q(u):
            for g in range(CH // 16):
                qidx[u][pl.ds(g * 16, 16)] = idxb[u][pl.ds(g * 16, 16)] >> 2

        b_idx(0, 0).start()
        b_idx(1, 1).start()
        b_idx(0, 0).wait()
        make_q(0)
        b_gather(0, 0).start()

        def c_body(t, _):
            for u in (0, 1):
                c = 2 * t + u
                b_gather(c, u).wait()

                @pl.when(c + 1 < NCHUNK)
                def _():
                    b_idx(c + 1, 1 - u).wait()
                    make_q(1 - u)
                    b_gather(c + 1, 1 - u).start()

                @pl.when(c >= 2)
                def _():
                    b_store(c - 2, u).wait()

                for g in range(CH // 16):
                    kv = lane + g * 16
                    mv = (idxb[u][pl.ds(g * 16, 16)] & 3) * D
                    for e in range(D):
                        vals = plsc.load_gather(rows[u], [kv, mv + e])
                        plsc.store_scatter(
                            vout[u], [jnp.full((16,), e, jnp.int32), kv], vals)

                b_store(c, u).start()
                @pl.when(c + 2 < NCHUNK)
                def _():
                    b_idx(c + 2, u).start()
            return ()

        lax.fori_loop(0, NCHUNK // 2, c_body, (), unroll=False)
        b_store(NCHUNK - 2, 0).wait()
        b_store(NCHUNK - 1, 1).wait()

    return emb(flat_idx, tabT)


def kernel(x, table):
    flat = x.T.reshape(B * F)
    tabT = table.T
    outT = _emb_call(flat, tabT)
    return outT.transpose(2, 0, 1)


# speedup vs baseline: 1.3395x; 1.3395x over previous
import functools

import jax
import jax.numpy as jnp
from jax import lax
from jax.experimental import pallas as pl
from jax.experimental.pallas import tpu as pltpu
from jax.experimental.pallas import tpu_sc as plsc

_NUM_CORES = 2
_NUM_SUBCORES = 16
_NUM_WORKERS = _NUM_CORES * _NUM_SUBCORES


def _emb_lookup(flat_idx, table, *, chunk, n_chunks):
    b_per_w = chunk * n_chunks
    total = b_per_w * _NUM_WORKERS
    d = table.shape[1]
    mesh = plsc.VectorSubcoreMesh(core_axis_name="c", subcore_axis_name="s")

    @functools.partial(
        pl.kernel,
        out_type=jax.ShapeDtypeStruct((total, d), table.dtype),
        mesh=mesh,
        compiler_params=pltpu.CompilerParams(use_tc_tiling_on_sc=False),
        scratch_types=[
            pltpu.VMEM((chunk,), jnp.int32),
            pltpu.VMEM((chunk,), jnp.int32),
            pltpu.VMEM((chunk, d), table.dtype),
            pltpu.VMEM((chunk, d), table.dtype),
            pltpu.SemaphoreType.DMA,
            pltpu.SemaphoreType.DMA,
            pltpu.SemaphoreType.DMA,
            pltpu.SemaphoreType.DMA,
            pltpu.SemaphoreType.DMA,
            pltpu.SemaphoreType.DMA,
        ],
    )
    def emb(idx_hbm, table_hbm, out_hbm, i0, i1, r0, r1,
            si0, si1, sg0, sg1, ss0, ss1):
        wid = lax.axis_index("s") * _NUM_CORES + lax.axis_index("c")
        base = wid * b_per_w
        idx_bufs, row_bufs = [i0, i1], [r0, r1]
        sem_i, sem_g, sem_s = [si0, si1], [sg0, sg1], [ss0, ss1]

        def idx_load(c):
            b = c % 2
            return pltpu.make_async_copy(
                idx_hbm.at[pl.ds(base + c * chunk, chunk)], idx_bufs[b],
                sem_i[b])

        def gather(c):
            b = c % 2
            return pltpu.make_async_copy(
                table_hbm.at[idx_bufs[b]], row_bufs[b], sem_g[b])

        def store(c):
            b = c % 2
            return pltpu.make_async_copy(
                row_bufs[b], out_hbm.at[pl.ds(base + c * chunk, chunk)],
                sem_s[b])

        idx_load(0).start()
        if n_chunks > 1:
            idx_load(1).start()
        idx_load(0).wait()
        gather(0).start()

        for c in range(n_chunks):
            if c + 1 < n_chunks:
                if c >= 1:
                    store(c - 1).wait()
                idx_load(c + 1).wait()
                gather(c + 1).start()
            gather(c).wait()
            store(c).start()
            if c + 2 < n_chunks:
                idx_load(c + 2).start()

        if n_chunks > 1:
            store(n_chunks - 2).wait()
        store(n_chunks - 1).wait()

    return emb(flat_idx, table)


def kernel(x, table):
    b, f = x.shape
    v, d = table.shape
    total = b * f
    assert total % _NUM_WORKERS == 0
    b_per_w = total // _NUM_WORKERS
    n_chunks = 8
    assert b_per_w % n_chunks == 0
    chunk = b_per_w // n_chunks

    flat = x.reshape(total).astype(jnp.int32)
    out = _emb_lookup(flat, table, chunk=chunk, n_chunks=n_chunks)
    return out.reshape(b, f, d)
